# baseline (speedup 1.0000x reference)
import functools

import jax
import jax.numpy as jnp
import numpy as np
from jax import lax
from jax.experimental import pallas as pl
from jax.experimental.pallas import tpu as pltpu
from jax.experimental.pallas import tpu_sc as plsc

NB = 8
N = 2048
NPAIR = 3
NTASK = 24
SC_TASKS = 16
TC_TASKS = NTASK - SC_TASKS
NCHUNK = SC_TASKS * 4
QI = 512
IBU = 2
TILES = N // 256
ROW_TILES = QI * 32 // 256


_INF = float(np.inf)


def _mesh():
    return plsc.VectorSubcoreMesh(core_axis_name="c", subcore_axis_name="s")


def _task_sets(gt):
    p = gt // NB
    b = gt % NB
    px = p // 2
    py = (p + 3) // 2
    return (px * NB + b) * 2, (py * NB + b) * 2


def _stage_sc(xsp, abf):

    @functools.partial(
        pl.kernel,
        out_type=jax.ShapeDtypeStruct((NCHUNK, ROW_TILES + TILES, 2, 128),
                                      jnp.bfloat16),
        mesh=_mesh(),
        scratch_types=[
            pltpu.VMEM((QI // 8, 2, 128), jnp.bfloat16),
            pltpu.VMEM((QI // 8, 2, 128), jnp.bfloat16),
            pltpu.VMEM((TILES, 2, 128), jnp.bfloat16),
            pltpu.VMEM((TILES, 2, 128), jnp.bfloat16),
            pltpu.VMEM((TILES, 2, 128), jnp.bfloat16),
            pltpu.VMEM((ROW_TILES, 2, 128), jnp.bfloat16),
        ],
    )
    def k(xsp_hbm, abf_hbm, out_hbm, x0b, x1b, y0b, y1b, cmb, rwb):
        wid = lax.axis_index("c") * 16 + lax.axis_index("s")
        for kk in range(NCHUNK // 32):
            ch = wid * (NCHUNK // 32) + kk
            t = ch // 4
            qt = ch % 4
            xrow, yrow = _task_sets(t)

            pltpu.sync_copy(
                xsp_hbm.at[xrow, pl.ds(qt * (QI // 8), QI // 8)], x0b)
            pltpu.sync_copy(
                xsp_hbm.at[xrow + 1, pl.ds(qt * (QI // 8), QI // 8)], x1b)
            pltpu.sync_copy(abf_hbm.at[yrow], y0b)
            pltpu.sync_copy(abf_hbm.at[yrow + 1], y1b)

            inf216 = jnp.full((2, 16), _INF, jnp.bfloat16)

            def init_body(jt, _):
                for h in range(8):
                    cmb[jt, :, pl.ds(h * 16, 16)] = inf216
                return 0
            lax.fori_loop(0, TILES, init_body, 0)

            def ig_body(ig, _):
                for u in range(0, 16, IBU):
                    xs = []
                    for w in range(IBU):
                        i = ig * 16 + u + w
                        xsl = (i // 8, slice(None), pl.ds((i % 8) * 16, 16))
                        xs.append((x0b[xsl], x1b[xsl]))

                    def j_body(jt, accs):
                        accs = [list(av) for av in accs]
                        for h in range(8):
                            sl = (jt, slice(None), pl.ds(h * 16, 16))
                            yv0 = y0b[sl]
                            yv1 = y1b[sl]
                            cmv = cmb[sl]
                            for w in range(IBU):
                                d0 = yv0 - xs[w][0]
                                d1 = yv1 - xs[w][1]
                                dsq = d0 * d0 + d1 * d1
                                accs[w][h] = jnp.minimum(accs[w][h], dsq)
                                cmv = jnp.minimum(cmv, dsq)
                            cmb[sl] = cmv
                        return tuple(tuple(av) for av in accs)

                    accs = lax.fori_loop(0, TILES, j_body,
                                         ((inf216,) * 8,) * IBU)
                    for w in range(IBU):
                        m = accs[w][0]
                        for h in range(1, 8):
                            m = jnp.minimum(m, accs[w][h])
                        i = ig * 16 + u + w
                        rwb[i // 8, :, pl.ds((i % 8) * 16, 16)] = m
                return 0

            lax.fori_loop(0, QI // 16, ig_body, 0)
            pltpu.sync_copy(rwb, out_hbm.at[ch, pl.ds(0, ROW_TILES)])
            pltpu.sync_copy(cmb, out_hbm.at[ch, pl.ds(ROW_TILES, TILES)])

    return k(xsp, abf)


def _two_lane(v0, v1):
    r = lax.broadcasted_iota(jnp.int32, (8, 128), 0)
    l = lax.broadcasted_iota(jnp.int32, (8, 128), 1)
    zero = jnp.zeros((8, 128), jnp.float32)
    return jnp.where((r == 0) & (l == 0), v0,
                     jnp.where((r == 0) & (l == 1), v1, zero))


def _tc_chamfer_body(xref, yref, oref):
    yA = yref[0]

    def body(k, carry):
        cm8, rs8a, rs8b = carry
        X8a = xref[0, pl.ds(k * 16, 8), :]
        X8b = xref[0, pl.ds(k * 16 + 8, 8), :]
        ua = lax.dot_general(X8a, yA, (((1,), (0,)), ((), ())),
                             precision=lax.Precision.HIGHEST,
                             preferred_element_type=jnp.float32)
        ub = lax.dot_general(X8b, yA, (((1,), (0,)), ((), ())),
                             precision=lax.Precision.HIGHEST,
                             preferred_element_type=jnp.float32)
        rs8a = rs8a + jnp.sqrt(jnp.maximum(jnp.min(ua, axis=1), 0.0))
        rs8b = rs8b + jnp.sqrt(jnp.maximum(jnp.min(ub, axis=1), 0.0))
        cm8 = jnp.minimum(cm8, jnp.minimum(ua, ub))
        return cm8, rs8a, rs8b

    cm8, rs8a, rs8b = lax.fori_loop(
        0, N // 16, body,
        (jnp.full((8, N), _INF, jnp.float32),
         jnp.zeros((8,), jnp.float32), jnp.zeros((8,), jnp.float32)))
    colsum = jnp.sum(jnp.sqrt(jnp.maximum(jnp.min(cm8, axis=0), 0.0)))
    oref[0] = _two_lane(jnp.sum(rs8a) + jnp.sum(rs8b), colsum)


def _tc_chamfer(xaug, yaug):
    def xmap(t):
        return (_task_sets(t + SC_TASKS)[0] // 2, 0, 0)

    def ymap(t):
        return (_task_sets(t + SC_TASKS)[1] // 2, 0, 0)

    return pl.pallas_call(
        _tc_chamfer_body,
        grid=(TC_TASKS,),
        in_specs=[pl.BlockSpec((1, N, 4), xmap),
                  pl.BlockSpec((1, 4, N), ymap)],
        out_specs=pl.BlockSpec((1, 8, 128), lambda t: (t, 0, 0)),
        out_shape=jax.ShapeDtypeStruct((TC_TASKS, 8, 128), jnp.float32),
    )(xaug, yaug)


def _tc_reduce_body(rref, oref):
    rows = rref[:, :ROW_TILES].astype(jnp.float32)
    rm = jnp.min(rows.reshape(4, ROW_TILES, 2, 8, 16), axis=(2, 4))
    cols = rref[:, ROW_TILES:].astype(jnp.float32)
    cm = jnp.min(cols, axis=0)
    oref[0] = _two_lane(jnp.sum(jnp.sqrt(rm)), jnp.sum(jnp.sqrt(cm)))


def _tc_reduce(raw):
    return pl.pallas_call(
        _tc_reduce_body,
        grid=(SC_TASKS,),
        in_specs=[pl.BlockSpec((4, ROW_TILES + TILES, 2, 128),
                               lambda t: (t, 0, 0, 0))],
        out_specs=pl.BlockSpec((1, 8, 128), lambda t: (t, 0, 0)),
        out_shape=jax.ShapeDtypeStruct((SC_TASKS, 8, 128), jnp.float32),
    )(raw)


def kernel(point_set1, point_set2, point_set3):
    a = jnp.stack([point_set1, point_set2, point_set3])
    a = jnp.transpose(a, (0, 1, 3, 2))

    xbf = a.astype(jnp.bfloat16).reshape(3 * NB * 2, N)
    abf = xbf.reshape(3 * NB * 2, TILES, 2, 128)
    xsp = jnp.broadcast_to(xbf.reshape(3 * NB * 2, N // 8, 1, 8, 1),
                           (3 * NB * 2, N // 8, 2, 8, 16))
    xsp = xsp.reshape(3 * NB * 2, N // 8, 2, 128)

    a3 = jnp.stack([point_set1, point_set2, point_set3]).reshape(
        3 * NB, N, 2)
    nrm = jnp.sum(a3 * a3, axis=-1, keepdims=True)
    ones = jnp.ones_like(nrm)
    xaug = jnp.concatenate([a3, nrm, ones], axis=-1)
    yaug = jnp.concatenate([-2.0 * a3, ones, nrm],
                           axis=-1).transpose(0, 2, 1)
    raw = _stage_sc(xsp, abf)
    tc_sums = _tc_chamfer(xaug, yaug)[:, 0, :2]
    sc_sums = _tc_reduce(raw)[:, 0, :2]

    sums = jnp.concatenate([sc_sums, tc_sums], axis=0)
    dist = sums.sum(-1).reshape(NPAIR, NB) / (2.0 * N)
    return jnp.mean(1.0 - dist, axis=0)


# speedup vs baseline: 2.6396x; 2.6396x over previous
import functools

import jax
import jax.numpy as jnp
import numpy as np
from jax import lax
from jax.experimental import pallas as pl
from jax.experimental.pallas import tpu as pltpu
from jax.experimental.pallas import tpu_sc as plsc

NB = 8
N = 2048
NPAIR = 3
NTASK = 24
SC_TASKS = 16
TC_TASKS = NTASK - SC_TASKS
NCHUNK = SC_TASKS * 4
QI = 512
IBU = 2
TILES = N // 256
ROW_TILES = QI * 32 // 256


_INF = float(np.inf)


def _mesh():
    return plsc.VectorSubcoreMesh(core_axis_name="c", subcore_axis_name="s")


def _task_sets(gt):
    p = gt // NB
    b = gt % NB
    px = p // 2
    py = (p + 3) // 2
    return (px * NB + b) * 2, (py * NB + b) * 2


def _stage_sc(xsp, abf):

    @functools.partial(
        pl.kernel,
        out_type=jax.ShapeDtypeStruct((NCHUNK, ROW_TILES + TILES, 2, 128),
                                      jnp.bfloat16),
        mesh=_mesh(),
        scratch_types=[
            pltpu.VMEM((QI // 8, 2, 128), jnp.bfloat16),
            pltpu.VMEM((QI // 8, 2, 128), jnp.bfloat16),
            pltpu.VMEM((TILES, 2, 128), jnp.bfloat16),
            pltpu.VMEM((TILES, 2, 128), jnp.bfloat16),
            pltpu.VMEM((TILES, 2, 128), jnp.bfloat16),
            pltpu.VMEM((ROW_TILES, 2, 128), jnp.bfloat16),
        ],
    )
    def k(xsp_hbm, abf_hbm, out_hbm, x0b, x1b, y0b, y1b, cmb, rwb):
        wid = lax.axis_index("c") * 16 + lax.axis_index("s")
        for kk in range(NCHUNK // 32):
            ch = wid * (NCHUNK // 32) + kk
            t = ch // 4
            qt = ch % 4
            xrow, yrow = _task_sets(t)

            pltpu.sync_copy(
                xsp_hbm.at[xrow, pl.ds(qt * (QI // 8), QI // 8)], x0b)
            pltpu.sync_copy(
                xsp_hbm.at[xrow + 1, pl.ds(qt * (QI // 8), QI // 8)], x1b)
            pltpu.sync_copy(abf_hbm.at[yrow], y0b)
            pltpu.sync_copy(abf_hbm.at[yrow + 1], y1b)

            inf216 = jnp.full((2, 16), _INF, jnp.bfloat16)

            def init_body(jt, _):
                for h in range(8):
                    cmb[jt, :, pl.ds(h * 16, 16)] = inf216
                return 0
            lax.fori_loop(0, TILES, init_body, 0)

            def ig_body(ig, _):
                for u in range(0, 16, IBU):
                    xs = []
                    for w in range(IBU):
                        i = ig * 16 + u + w
                        xsl = (i // 8, slice(None), pl.ds((i % 8) * 16, 16))
                        xs.append((x0b[xsl], x1b[xsl]))

                    def j_body(jt, accs):
                        accs = [list(av) for av in accs]
                        for h in range(8):
                            sl = (jt, slice(None), pl.ds(h * 16, 16))
                            yv0 = y0b[sl]
                            yv1 = y1b[sl]
                            cmv = cmb[sl]
                            for w in range(IBU):
                                d0 = yv0 - xs[w][0]
                                d1 = yv1 - xs[w][1]
                                dsq = d0 * d0 + d1 * d1
                                accs[w][h] = jnp.minimum(accs[w][h], dsq)
                                cmv = jnp.minimum(cmv, dsq)
                            cmb[sl] = cmv
                        return tuple(tuple(av) for av in accs)

                    accs = lax.fori_loop(0, TILES, j_body,
                                         ((inf216,) * 8,) * IBU)
                    for w in range(IBU):
                        m = accs[w][0]
                        for h in range(1, 8):
                            m = jnp.minimum(m, accs[w][h])
                        i = ig * 16 + u + w
                        rwb[i // 8, :, pl.ds((i % 8) * 16, 16)] = m
                return 0

            lax.fori_loop(0, QI // 16, ig_body, 0)
            pltpu.sync_copy(rwb, out_hbm.at[ch, pl.ds(0, ROW_TILES)])
            pltpu.sync_copy(cmb, out_hbm.at[ch, pl.ds(ROW_TILES, TILES)])

    return k(xsp, abf)


def _two_lane(v0, v1):
    r = lax.broadcasted_iota(jnp.int32, (8, 128), 0)
    l = lax.broadcasted_iota(jnp.int32, (8, 128), 1)
    zero = jnp.zeros((8, 128), jnp.float32)
    return jnp.where((r == 0) & (l == 0), v0,
                     jnp.where((r == 0) & (l == 1), v1, zero))


def _tc_chamfer_body(xref, yref, oref):
    y0 = yref[0, 0:1, :]
    y1 = yref[0, 1:2, :]

    def one(k8):
        X8 = xref[0, pl.ds(k8 * 8, 8), :]
        d0 = X8[:, 0:1] - y0
        d1 = X8[:, 1:2] - y1
        return d0 * d0 + d1 * d1

    def body(k, carry):
        cm8, rs = carry
        us = [one(k * 4 + s) for s in range(4)]
        rs = tuple(rs[s] + jnp.sqrt(jnp.min(us[s], axis=1))
                   for s in range(4))
        cm8 = jnp.minimum(cm8, jnp.minimum(jnp.minimum(us[0], us[1]),
                                           jnp.minimum(us[2], us[3])))
        return cm8, rs

    z8 = jnp.zeros((8,), jnp.float32)
    cm8, rs = lax.fori_loop(
        0, N // 32, body,
        (jnp.full((8, N), _INF, jnp.float32), (z8, z8, z8, z8)))
    colsum = jnp.sum(jnp.sqrt(jnp.min(cm8, axis=0)))
    rowsum = jnp.sum(rs[0] + rs[1] + rs[2] + rs[3])
    oref[0] = _two_lane(rowsum, colsum)


def _tc_chamfer(a3, a2):
    def xmap(t):
        return (_task_sets(t + SC_TASKS)[0] // 2, 0, 0)

    def ymap(t):
        return (_task_sets(t + SC_TASKS)[1] // 2, 0, 0)

    return pl.pallas_call(
        _tc_chamfer_body,
        grid=(TC_TASKS,),
        in_specs=[pl.BlockSpec((1, N, 2), xmap),
                  pl.BlockSpec((1, 2, N), ymap)],
        out_specs=pl.BlockSpec((1, 8, 128), lambda t: (t, 0, 0)),
        out_shape=jax.ShapeDtypeStruct((TC_TASKS, 8, 128), jnp.float32),
    )(a3, a2)


def _tc_reduce_body(rref, oref):
    rows = rref[:, :ROW_TILES].astype(jnp.float32)
    rm = jnp.min(rows.reshape(4, ROW_TILES, 2, 8, 16), axis=(2, 4))
    cols = rref[:, ROW_TILES:].astype(jnp.float32)
    cm = jnp.min(cols, axis=0)
    oref[0] = _two_lane(jnp.sum(jnp.sqrt(rm)), jnp.sum(jnp.sqrt(cm)))


def _tc_reduce(raw):
    return pl.pallas_call(
        _tc_reduce_body,
        grid=(SC_TASKS,),
        in_specs=[pl.BlockSpec((4, ROW_TILES + TILES, 2, 128),
                               lambda t: (t, 0, 0, 0))],
        out_specs=pl.BlockSpec((1, 8, 128), lambda t: (t, 0, 0)),
        out_shape=jax.ShapeDtypeStruct((SC_TASKS, 8, 128), jnp.float32),
    )(raw)


def kernel(point_set1, point_set2, point_set3):
    a = jnp.stack([point_set1, point_set2, point_set3])
    a = jnp.transpose(a, (0, 1, 3, 2))

    xbf = a.astype(jnp.bfloat16).reshape(3 * NB * 2, N)
    abf = xbf.reshape(3 * NB * 2, TILES, 2, 128)
    xsp = jnp.broadcast_to(xbf.reshape(3 * NB * 2, N // 8, 1, 8, 1),
                           (3 * NB * 2, N // 8, 2, 8, 16))
    xsp = xsp.reshape(3 * NB * 2, N // 8, 2, 128)

    a3 = jnp.stack([point_set1, point_set2, point_set3]).reshape(
        3 * NB, N, 2)
    a2 = a.reshape(3 * NB, 2, N)
    raw = _stage_sc(xsp, abf)
    tc_sums = _tc_chamfer(a3, a2)[:, 0, :2]
    sc_sums = _tc_reduce(raw)[:, 0, :2]

    sums = jnp.concatenate([sc_sums, tc_sums], axis=0)
    dist = sums.sum(-1).reshape(NPAIR, NB) / (2.0 * N)
    return jnp.mean(1.0 - dist, axis=0)


# speedup vs baseline: 2.6418x; 1.0008x over previous
import functools

import jax
import jax.numpy as jnp
import numpy as np
from jax import lax
from jax.experimental import pallas as pl
from jax.experimental.pallas import tpu as pltpu
from jax.experimental.pallas import tpu_sc as plsc

NB = 8
N = 2048
NPAIR = 3
NTASK = 24
SC_TASKS = 16
TC_TASKS = NTASK - SC_TASKS
NCHUNK = SC_TASKS * 4
QI = 512
IBU = 2
TILES = N // 256
ROW_TILES = QI * 32 // 256
ROW_FLAT = ROW_TILES * 256


_INF = float(np.inf)


def _mesh():
    return plsc.VectorSubcoreMesh(core_axis_name="c", subcore_axis_name="s")


def _task_sets(gt):
    p = gt // NB
    b = gt % NB
    px = p // 2
    py = (p + 3) // 2
    return (px * NB + b) * 2, (py * NB + b) * 2


def _stage_sc(xsp, abf):

    @functools.partial(
        pl.kernel,
        out_type=jax.ShapeDtypeStruct((NCHUNK, ROW_TILES + TILES, 2, 128),
                                      jnp.bfloat16),
        mesh=_mesh(),
        scratch_types=[
            pltpu.VMEM((QI // 8, 2, 128), jnp.bfloat16),
            pltpu.VMEM((QI // 8, 2, 128), jnp.bfloat16),
            pltpu.VMEM((TILES, 2, 128), jnp.bfloat16),
            pltpu.VMEM((TILES, 2, 128), jnp.bfloat16),
            pltpu.VMEM((TILES, 2, 128), jnp.bfloat16),
            pltpu.VMEM((ROW_TILES, 2, 128), jnp.bfloat16),
        ],
    )
    def k(xsp_hbm, abf_hbm, out_hbm, x0b, x1b, y0b, y1b, cmb, rwb):
        wid = lax.axis_index("c") * 16 + lax.axis_index("s")
        for kk in range(NCHUNK // 32):
            ch = wid * (NCHUNK // 32) + kk
            t = ch // 4
            qt = ch % 4
            xrow, yrow = _task_sets(t)

            pltpu.sync_copy(
                xsp_hbm.at[xrow, pl.ds(qt * (QI // 8), QI // 8)], x0b)
            pltpu.sync_copy(
                xsp_hbm.at[xrow + 1, pl.ds(qt * (QI // 8), QI // 8)], x1b)
            pltpu.sync_copy(abf_hbm.at[yrow], y0b)
            pltpu.sync_copy(abf_hbm.at[yrow + 1], y1b)

            inf216 = jnp.full((2, 16), _INF, jnp.bfloat16)

            def init_body(jt, _):
                for h in range(8):
                    cmb[jt, :, pl.ds(h * 16, 16)] = inf216
                return 0
            lax.fori_loop(0, TILES, init_body, 0)

            def ig_body(ig, _):
                for u in range(0, 16, IBU):
                    xs = []
                    for w in range(IBU):
                        i = ig * 16 + u + w
                        xsl = (i // 8, slice(None), pl.ds((i % 8) * 16, 16))
                        xs.append((x0b[xsl], x1b[xsl]))

                    def j_body(jt, accs):
                        accs = [list(av) for av in accs]
                        for h in range(8):
                            sl = (jt, slice(None), pl.ds(h * 16, 16))
                            yv0 = y0b[sl]
                            yv1 = y1b[sl]
                            cmv = cmb[sl]
                            for w in range(IBU):
                                d0 = yv0 - xs[w][0]
                                d1 = yv1 - xs[w][1]
                                dsq = d0 * d0 + d1 * d1
                                accs[w][h] = jnp.minimum(accs[w][h], dsq)
                                cmv = jnp.minimum(cmv, dsq)
                            cmb[sl] = cmv
                        return tuple(tuple(av) for av in accs)

                    accs = lax.fori_loop(0, TILES, j_body,
                                         ((inf216,) * 8,) * IBU)
                    for w in range(IBU):
                        m = accs[w][0]
                        for h in range(1, 8):
                            m = jnp.minimum(m, accs[w][h])
                        i = ig * 16 + u + w
                        rwb[i // 8, :, pl.ds((i % 8) * 16, 16)] = m
                return 0

            lax.fori_loop(0, QI // 16, ig_body, 0)
            pltpu.sync_copy(rwb, out_hbm.at[ch, pl.ds(0, ROW_TILES)])
            pltpu.sync_copy(cmb, out_hbm.at[ch, pl.ds(ROW_TILES, TILES)])

    return k(xsp, abf)


def _two_lane(v0, v1):
    r = lax.broadcasted_iota(jnp.int32, (8, 128), 0)
    l = lax.broadcasted_iota(jnp.int32, (8, 128), 1)
    zero = jnp.zeros((8, 128), jnp.float32)
    return jnp.where((r == 0) & (l == 0), v0,
                     jnp.where((r == 0) & (l == 1), v1, zero))


def _tc_chamfer_body(xref, yref, oref):
    y0 = yref[0, 0:1, :]
    y1 = yref[0, 1:2, :]

    def one(k8):
        X8 = xref[0, pl.ds(k8 * 8, 8), :]
        d0 = X8[:, 0:1] - y0
        d1 = X8[:, 1:2] - y1
        return d0 * d0 + d1 * d1

    def body(k, carry):
        cm8, rs = carry
        us = [one(k * 4 + s) for s in range(4)]
        rs = tuple(rs[s] + jnp.sqrt(jnp.min(us[s], axis=1))
                   for s in range(4))
        cm8 = jnp.minimum(cm8, jnp.minimum(jnp.minimum(us[0], us[1]),
                                           jnp.minimum(us[2], us[3])))
        return cm8, rs

    z8 = jnp.zeros((8,), jnp.float32)
    cm8, rs = lax.fori_loop(
        0, N // 32, body,
        (jnp.full((8, N), _INF, jnp.float32), (z8, z8, z8, z8)))
    colsum = jnp.sum(jnp.sqrt(jnp.min(cm8, axis=0)))
    rowsum = jnp.sum(rs[0] + rs[1] + rs[2] + rs[3])
    oref[0] = _two_lane(rowsum, colsum)


def _tc_chamfer(a3, a2):
    def xmap(t):
        return (_task_sets(t + SC_TASKS)[0] // 2, 0, 0)

    def ymap(t):
        return (_task_sets(t + SC_TASKS)[1] // 2, 0, 0)

    return pl.pallas_call(
        _tc_chamfer_body,
        grid=(TC_TASKS,),
        in_specs=[pl.BlockSpec((1, N, 2), xmap),
                  pl.BlockSpec((1, 2, N), ymap)],
        out_specs=pl.BlockSpec((1, 8, 128), lambda t: (t, 0, 0)),
        out_shape=jax.ShapeDtypeStruct((TC_TASKS, 8, 128), jnp.float32),
    )(a3, a2)


def _tc_reduce_body(rref, oref):
    rows = rref[:, :ROW_TILES].astype(jnp.float32)
    rm = jnp.min(rows.reshape(4, ROW_TILES, 2, 8, 16), axis=(2, 4))
    cols = rref[:, ROW_TILES:].astype(jnp.float32)
    cm = jnp.min(cols, axis=0)
    oref[0] = _two_lane(jnp.sum(jnp.sqrt(rm)), jnp.sum(jnp.sqrt(cm)))


def _tc_reduce(raw):
    return pl.pallas_call(
        _tc_reduce_body,
        grid=(SC_TASKS,),
        in_specs=[pl.BlockSpec((4, ROW_TILES + TILES, 2, 128),
                               lambda t: (t, 0, 0, 0))],
        out_specs=pl.BlockSpec((1, 8, 128), lambda t: (t, 0, 0)),
        out_shape=jax.ShapeDtypeStruct((SC_TASKS, 8, 128), jnp.float32),
    )(raw)


def kernel(point_set1, point_set2, point_set3):
    a = jnp.stack([point_set1, point_set2, point_set3])
    a = jnp.transpose(a, (0, 1, 3, 2))

    xbf = a.astype(jnp.bfloat16).reshape(3 * NB * 2, N)
    abf = xbf.reshape(3 * NB * 2, TILES, 2, 128)
    xsp = jnp.broadcast_to(xbf.reshape(3 * NB * 2, N // 8, 1, 8, 1),
                           (3 * NB * 2, N // 8, 2, 8, 16))
    xsp = xsp.reshape(3 * NB * 2, N // 8, 2, 128)

    a3 = jnp.stack([point_set1, point_set2, point_set3]).reshape(
        3 * NB, N, 2)
    a2 = a.reshape(3 * NB, 2, N)
    raw = _stage_sc(xsp, abf)
    tc_sums = _tc_chamfer(a3, a2)[:, 0, :2]
    sc_sums = _tc_reduce(raw)[:, 0, :2]

    sums = jnp.concatenate([sc_sums, tc_sums], axis=0)
    dist = sums.sum(-1).reshape(NPAIR, NB) / (2.0 * N)
    return jnp.mean(1.0 - dist, axis=0)


# speedup vs baseline: 2.8673x; 1.0854x over previous
import functools

import jax
import jax.numpy as jnp
import numpy as np
from jax import lax
from jax.experimental import pallas as pl
from jax.experimental.pallas import tpu as pltpu
from jax.experimental.pallas import tpu_sc as plsc

NB = 8
N = 2048
NPAIR = 3
NTASK = 24
SC_TASKS = 16
TC_TASKS = NTASK - SC_TASKS
NCHUNK = SC_TASKS * 4
QI = 512
IBU = 2
TILES = N // 256
ROW_TILES = QI * 32 // 256
ROW_FLAT = ROW_TILES * 256


_INF = float(np.inf)


def _mesh():
    return plsc.VectorSubcoreMesh(core_axis_name="c", subcore_axis_name="s")


def _task_sets(gt):
    p = gt // NB
    b = gt % NB
    px = p // 2
    py = (p + 3) // 2
    return (px * NB + b) * 2, (py * NB + b) * 2


def _stage_sc(xsp, abf):

    @functools.partial(
        pl.kernel,
        out_type=jax.ShapeDtypeStruct((NCHUNK, ROW_TILES + TILES, 2, 128),
                                      jnp.bfloat16),
        mesh=_mesh(),
        scratch_types=[
            pltpu.VMEM((QI // 8, 2, 128), jnp.bfloat16),
            pltpu.VMEM((QI // 8, 2, 128), jnp.bfloat16),
            pltpu.VMEM((TILES, 2, 128), jnp.bfloat16),
            pltpu.VMEM((TILES, 2, 128), jnp.bfloat16),
            pltpu.VMEM((TILES, 2, 128), jnp.bfloat16),
            pltpu.VMEM((ROW_TILES, 2, 128), jnp.bfloat16),
        ],
    )
    def k(xsp_hbm, abf_hbm, out_hbm, x0b, x1b, y0b, y1b, cmb, rwb):
        wid = lax.axis_index("c") * 16 + lax.axis_index("s")
        for kk in range(NCHUNK // 32):
            ch = wid * (NCHUNK // 32) + kk
            t = ch // 4
            qt = ch % 4
            xrow, yrow = _task_sets(t)

            pltpu.sync_copy(
                xsp_hbm.at[xrow, pl.ds(qt * (QI // 8), QI // 8)], x0b)
            pltpu.sync_copy(
                xsp_hbm.at[xrow + 1, pl.ds(qt * (QI // 8), QI // 8)], x1b)
            pltpu.sync_copy(abf_hbm.at[yrow], y0b)
            pltpu.sync_copy(abf_hbm.at[yrow + 1], y1b)

            inf216 = jnp.full((2, 16), _INF, jnp.bfloat16)

            def init_body(jt, _):
                for h in range(8):
                    cmb[jt, :, pl.ds(h * 16, 16)] = inf216
                return 0
            lax.fori_loop(0, TILES, init_body, 0)

            def ig_body(ig, _):
                for u in range(0, 16, IBU):
                    xs = []
                    for w in range(IBU):
                        i = ig * 16 + u + w
                        xsl = (i // 8, slice(None), pl.ds((i % 8) * 16, 16))
                        xs.append((x0b[xsl], x1b[xsl]))

                    def j_body(jt, accs):
                        accs = [list(av) for av in accs]
                        for h in range(8):
                            sl = (jt, slice(None), pl.ds(h * 16, 16))
                            yv0 = y0b[sl]
                            yv1 = y1b[sl]
                            cmv = cmb[sl]
                            for w in range(IBU):
                                d0 = yv0 - xs[w][0]
                                d1 = yv1 - xs[w][1]
                                dsq = d0 * d0 + d1 * d1
                                accs[w][h] = jnp.minimum(accs[w][h], dsq)
                                cmv = jnp.minimum(cmv, dsq)
                            cmb[sl] = cmv
                        return tuple(tuple(av) for av in accs)

                    accs = lax.fori_loop(0, TILES, j_body,
                                         ((inf216,) * 8,) * IBU)
                    for w in range(IBU):
                        m = accs[w][0]
                        for h in range(1, 8):
                            m = jnp.minimum(m, accs[w][h])
                        i = ig * 16 + u + w
                        rwb[i // 8, :, pl.ds((i % 8) * 16, 16)] = m
                return 0

            lax.fori_loop(0, QI // 16, ig_body, 0)
            pltpu.sync_copy(rwb, out_hbm.at[ch, pl.ds(0, ROW_TILES)])
            pltpu.sync_copy(cmb, out_hbm.at[ch, pl.ds(ROW_TILES, TILES)])

    return k(xsp, abf)


def _two_lane(v0, v1):
    r = lax.broadcasted_iota(jnp.int32, (8, 128), 0)
    l = lax.broadcasted_iota(jnp.int32, (8, 128), 1)
    zero = jnp.zeros((8, 128), jnp.float32)
    return jnp.where((r == 0) & (l == 0), v0,
                     jnp.where((r == 0) & (l == 1), v1, zero))


def _tc_chamfer_body(xref, yref, oref):
    y0 = yref[0, 0:1, :]
    y1 = yref[0, 1:2, :]

    def one(k8):
        X8 = xref[0, pl.ds(k8 * 8, 8), :]
        d0 = X8[:, 0:1] - y0
        d1 = X8[:, 1:2] - y1
        return d0 * d0 + d1 * d1

    def body(k, carry):
        cm8, rs = carry
        us = [one(k * 8 + s) for s in range(8)]
        rs = tuple(rs[s] + jnp.sqrt(jnp.min(us[s], axis=1))
                   for s in range(8))
        m4 = [jnp.minimum(us[2 * s], us[2 * s + 1]) for s in range(4)]
        cm8 = jnp.minimum(cm8, jnp.minimum(jnp.minimum(m4[0], m4[1]),
                                           jnp.minimum(m4[2], m4[3])))
        return cm8, rs

    z8 = jnp.zeros((8,), jnp.float32)
    cm8, rs = lax.fori_loop(
        0, N // 64, body,
        (jnp.full((8, N), _INF, jnp.float32), (z8,) * 8))
    colsum = jnp.sum(jnp.sqrt(jnp.min(cm8, axis=0)))
    rowsum = jnp.sum(sum(rs[1:], rs[0]))
    oref[0] = _two_lane(rowsum, colsum)


def _tc_chamfer(a3, a2):
    def xmap(t):
        return (_task_sets(t + SC_TASKS)[0] // 2, 0, 0)

    def ymap(t):
        return (_task_sets(t + SC_TASKS)[1] // 2, 0, 0)

    return pl.pallas_call(
        _tc_chamfer_body,
        grid=(TC_TASKS,),
        in_specs=[pl.BlockSpec((1, N, 2), xmap),
                  pl.BlockSpec((1, 2, N), ymap)],
        out_specs=pl.BlockSpec((1, 8, 128), lambda t: (t, 0, 0)),
        out_shape=jax.ShapeDtypeStruct((TC_TASKS, 8, 128), jnp.float32),
    )(a3, a2)


def _tc_reduce_body(rref, oref):
    rows = rref[:, :ROW_TILES].astype(jnp.float32)
    rm = jnp.min(rows.reshape(4, ROW_TILES, 2, 8, 16), axis=(2, 4))
    cols = rref[:, ROW_TILES:].astype(jnp.float32)
    cm = jnp.min(cols, axis=0)
    oref[0] = _two_lane(jnp.sum(jnp.sqrt(rm)), jnp.sum(jnp.sqrt(cm)))


def _tc_reduce(raw):
    return pl.pallas_call(
        _tc_reduce_body,
        grid=(SC_TASKS,),
        in_specs=[pl.BlockSpec((4, ROW_TILES + TILES, 2, 128),
                               lambda t: (t, 0, 0, 0))],
        out_specs=pl.BlockSpec((1, 8, 128), lambda t: (t, 0, 0)),
        out_shape=jax.ShapeDtypeStruct((SC_TASKS, 8, 128), jnp.float32),
    )(raw)


def kernel(point_set1, point_set2, point_set3):
    a = jnp.stack([point_set1, point_set2, point_set3])
    a = jnp.transpose(a, (0, 1, 3, 2))

    xbf = a.astype(jnp.bfloat16).reshape(3 * NB * 2, N)
    abf = xbf.reshape(3 * NB * 2, TILES, 2, 128)
    xsp = jnp.broadcast_to(xbf.reshape(3 * NB * 2, N // 8, 1, 8, 1),
                           (3 * NB * 2, N // 8, 2, 8, 16))
    xsp = xsp.reshape(3 * NB * 2, N // 8, 2, 128)

    a3 = jnp.stack([point_set1, point_set2, point_set3]).reshape(
        3 * NB, N, 2)
    a2 = a.reshape(3 * NB, 2, N)
    raw = _stage_sc(xsp, abf)
    tc_sums = _tc_chamfer(a3, a2)[:, 0, :2]
    sc_sums = _tc_reduce(raw)[:, 0, :2]

    sums = jnp.concatenate([sc_sums, tc_sums], axis=0)
    dist = sums.sum(-1).reshape(NPAIR, NB) / (2.0 * N)
    return jnp.mean(1.0 - dist, axis=0)
